# baseline (speedup 1.0000x reference)
import functools

import jax
import jax.numpy as jnp
from jax import lax
from jax.experimental import pallas as pl
from jax.experimental.pallas import tpu as pltpu
from jax.experimental.pallas import tpu_sc as plsc

NC = 2
NS = 16
NW = NC * NS
C = 80
DW = 16

_SC_PARAMS = pltpu.CompilerParams(use_tc_tiling_on_sc=False)


def _mesh():
    return plsc.VectorSubcoreMesh(
        core_axis_name="c", subcore_axis_name="s",
        num_cores=NC, num_subcores=NS)


def _sc_deg(dstp, n_pad, chunks):
    rpt = n_pad // NS

    @functools.partial(
        pl.kernel,
        out_type=jax.ShapeDtypeStruct((NC, n_pad, DW), jnp.float32),
        mesh=_mesh(),
        compiler_params=_SC_PARAMS,
        scratch_types=[
            pltpu.VMEM_SHARED((n_pad, DW), jnp.float32),
            pltpu.VMEM((chunks, C), jnp.int32),
            pltpu.VMEM((C, DW), jnp.float32),
            pltpu.VMEM((rpt, DW), jnp.float32),
        ])
    def deg_kernel(dst_hbm, out_hbm, acc, dstv, ones_b, obuf):
        c = lax.axis_index("c")
        s = lax.axis_index("s")
        wid = s * NC + c
        pltpu.sync_copy(dst_hbm.at[wid], dstv)

        def fill_ones(i, carry):
            ones_b[i, :] = jnp.ones((DW,), jnp.float32)
            return carry

        lax.fori_loop(0, C, fill_ones, 0)

        def fill_zero(i, carry):
            obuf[i, :] = jnp.zeros((DW,), jnp.float32)
            return carry

        lax.fori_loop(0, rpt, fill_zero, 0)
        pltpu.sync_copy(obuf, acc.at[pl.ds(s * rpt, rpt)])
        plsc.subcore_barrier()

        def body(k, carry):
            pltpu.sync_copy(ones_b, acc.at[dstv.at[k]], add=True)
            return carry

        lax.fori_loop(0, chunks, body, 0)
        plsc.subcore_barrier()
        pltpu.sync_copy(acc.at[pl.ds(s * rpt, rpt)], obuf)
        pltpu.sync_copy(obuf, out_hbm.at[c, pl.ds(s * rpt, rpt)])

    return deg_kernel(dstp)


_R = 4
_H = 16


def _sc_scatter(y, srcp, dstp, n_pad, chunks, d_h):
    rpt = n_pad // NS

    @functools.partial(
        pl.kernel,
        out_type=jax.ShapeDtypeStruct((NC, n_pad, d_h), jnp.float32),
        mesh=_mesh(),
        compiler_params=_SC_PARAMS,
        scratch_types=[
            pltpu.VMEM_SHARED((n_pad, d_h), jnp.float32),
            pltpu.VMEM((2, _H, C), jnp.int32),
            pltpu.VMEM((2, _H, C), jnp.int32),
            [pltpu.VMEM((C, d_h), jnp.float32) for _ in range(_R)],
            [pltpu.SemaphoreType.DMA for _ in range(_R)],
            [pltpu.SemaphoreType.DMA for _ in range(_R)],
            pltpu.SemaphoreType.DMA,
        ])
    def scat_kernel(y_hbm, src_hbm, dst_hbm, out_hbm, acc,
                    srcv, dstv, rows, gsem, ssem, isem):
        c = lax.axis_index("c")
        s = lax.axis_index("s")
        wid = s * NC + c

        def fill_zero(i, carry):
            for j in range(d_h // 16):
                rows[0][i, pl.ds(j * 16, 16)] = jnp.zeros((16,), jnp.float32)
            return carry

        lax.fori_loop(0, C, fill_zero, 0)
        off = 0
        while off < rpt:
            sz = min(C, rpt - off)
            pltpu.sync_copy(rows[0].at[pl.ds(0, sz)],
                            acc.at[pl.ds(s * rpt + off, sz)])
            off += sz

        def load_half(half, start, sync):
            hh = min(_H, chunks - start)
            dsrc = pltpu.async_copy(src_hbm.at[wid, pl.ds(start, hh)],
                                    srcv.at[half, pl.ds(0, hh)], isem)
            ddst = pltpu.async_copy(dst_hbm.at[wid, pl.ds(start, hh)],
                                    dstv.at[half, pl.ds(0, hh)], isem)
            if sync:
                dsrc.wait()
                ddst.wait()
            return dsrc, ddst

        load_half(0, 0, True)
        pend_idx = load_half(1, _H, False) if chunks > _H else None
        plsc.subcore_barrier()

        def gather(k):
            return pltpu.async_copy(
                y_hbm.at[srcv.at[(k // _H) % 2, k % _H]],
                rows[k % _R], gsem[k % _R])

        ahead = _R - 2 if _R > 2 else 1
        d_g = [None] * _R
        d_s = [None] * _R
        for j in range(min(ahead, chunks)):
            d_g[j] = gather(j)
        for k in range(chunks):
            b = k % _R
            if k + ahead - _R >= 0:
                d_s[(k + ahead - _R) % _R].wait()
            if k % _H == 2 and k > _H:
                nxt_start = (k // _H + 1) * _H
                if nxt_start < chunks:
                    pend_idx = load_half((k // _H + 1) % 2, nxt_start, False)
            if k + ahead < chunks:
                if (k + ahead) % _H == 0 and pend_idx is not None:
                    pend_idx[0].wait()
                    pend_idx[1].wait()
                    pend_idx = None
                d_g[(k + ahead) % _R] = gather(k + ahead)
            d_g[b].wait()
            d_s[b] = pltpu.async_copy(
                rows[b], acc.at[dstv.at[(k // _H) % 2, k % _H]],
                ssem[b], add=True)
        for j in range(max(0, chunks + ahead - _R), chunks):
            d_s[j % _R].wait()
        plsc.subcore_barrier()
        pltpu.sync_copy(acc.at[pl.ds(s * rpt, rpt)],
                        out_hbm.at[c, pl.ds(s * rpt, rpt)])

    return scat_kernel(y, srcp, dstp)


_BR = 256


def _tc_matmul(x_pad, W1, n_pad, d_in, d_h):
    def kfn(x_ref, w_ref, o_ref):
        o_ref[...] = jnp.dot(x_ref[...], w_ref[...],
                             preferred_element_type=jnp.float32)

    return pl.pallas_call(
        kfn,
        grid=(n_pad // _BR,),
        in_specs=[
            pl.BlockSpec((_BR, d_in), lambda i: (i, 0)),
            pl.BlockSpec((d_in, d_h), lambda i: (0, 0)),
        ],
        out_specs=pl.BlockSpec((_BR, d_h), lambda i: (i, 0)),
        out_shape=jax.ShapeDtypeStruct((n_pad, d_h), jnp.float32),
    )(x_pad, W1)


def _tc_scale(xw, degp, n_pad, d_h):
    def kfn(x_ref, d_ref, y_ref, dis_ref):
        dblk = d_ref[...]
        deg = dblk[0, :, 0] + dblk[1, :, 0] + 1.0
        dis = lax.rsqrt(deg)[:, None]
        y_ref[...] = x_ref[...] * dis
        dis_ref[...] = dis

    return pl.pallas_call(
        kfn,
        grid=(n_pad // _BR,),
        in_specs=[
            pl.BlockSpec((_BR, d_h), lambda i: (i, 0)),
            pl.BlockSpec((NC, _BR, DW), lambda i: (0, i, 0)),
        ],
        out_specs=[
            pl.BlockSpec((_BR, d_h), lambda i: (i, 0)),
            pl.BlockSpec((_BR, 1), lambda i: (i, 0)),
        ],
        out_shape=[
            jax.ShapeDtypeStruct((n_pad, d_h), jnp.float32),
            jax.ShapeDtypeStruct((n_pad, 1), jnp.float32),
        ],
    )(xw, degp)


def _tc_mid(aggp, y1, dis, b1r, W2, n_pad, d_h):
    def kfn(a_ref, y_ref, d_ref, b_ref, w_ref, o_ref):
        a = a_ref[0] + a_ref[1] + y_ref[...]
        dis = d_ref[...]
        h = jnp.maximum(a * dis + b_ref[...], 0.0)
        o_ref[...] = jnp.dot(h, w_ref[...],
                             preferred_element_type=jnp.float32) * dis

    return pl.pallas_call(
        kfn,
        grid=(n_pad // _BR,),
        in_specs=[
            pl.BlockSpec((NC, _BR, d_h), lambda i: (0, i, 0)),
            pl.BlockSpec((_BR, d_h), lambda i: (i, 0)),
            pl.BlockSpec((_BR, 1), lambda i: (i, 0)),
            pl.BlockSpec((1, d_h), lambda i: (0, 0)),
            pl.BlockSpec((d_h, d_h), lambda i: (0, 0)),
        ],
        out_specs=pl.BlockSpec((_BR, d_h), lambda i: (i, 0)),
        out_shape=jax.ShapeDtypeStruct((n_pad, d_h), jnp.float32),
    )(aggp, y1, dis, b1r, W2)


def _tc_final(aggp, y2, dis, b2r, n_pad, d_h):
    def kfn(a_ref, y_ref, d_ref, b_ref, o_ref):
        a = a_ref[0] + a_ref[1] + y_ref[...]
        o_ref[...] = jnp.maximum(a * d_ref[...] + b_ref[...], 0.0)

    return pl.pallas_call(
        kfn,
        grid=(n_pad // _BR,),
        in_specs=[
            pl.BlockSpec((NC, _BR, d_h), lambda i: (0, i, 0)),
            pl.BlockSpec((_BR, d_h), lambda i: (i, 0)),
            pl.BlockSpec((_BR, 1), lambda i: (i, 0)),
            pl.BlockSpec((1, d_h), lambda i: (0, 0)),
        ],
        out_specs=pl.BlockSpec((_BR, d_h), lambda i: (i, 0)),
        out_shape=jax.ShapeDtypeStruct((n_pad, d_h), jnp.float32),
    )(aggp, y2, dis, b2r)


def kernel(x, edge_index, W1, b1, W2, b2):
    n, d_in = x.shape
    e = edge_index.shape[1]
    d_h = W1.shape[1]

    chunks = -(-e // (NW * C))
    e_pad = NW * chunks * C
    rpt = 128 * (-(-(n + 1) // (NS * 128)))
    n_pad = NS * rpt

    src = jnp.pad(edge_index[0], (0, e_pad - e), constant_values=n)
    dst = jnp.pad(edge_index[1], (0, e_pad - e), constant_values=n)
    srcp = src.reshape(NW, chunks, C)
    dstp = dst.reshape(NW, chunks, C)
    x_pad = jnp.pad(x, ((0, n_pad - n), (0, 0)))
    b1r = b1.reshape(1, d_h)
    b2r = b2.reshape(1, d_h)

    xw1 = _tc_matmul(x_pad, W1, n_pad, d_in, d_h)
    degp = _sc_deg(dstp, n_pad, chunks)
    y1, dis = _tc_scale(xw1, degp, n_pad, d_h)
    agg1 = _sc_scatter(y1, srcp, dstp, n_pad, chunks, d_h)
    y2 = _tc_mid(agg1, y1, dis, b1r, W2, n_pad, d_h)
    agg2 = _sc_scatter(y2, srcp, dstp, n_pad, chunks, d_h)
    h2 = _tc_final(agg2, y2, dis, b2r, n_pad, d_h)
    return h2[:n]


# speedup vs baseline: 1.0227x; 1.0227x over previous
import functools

import jax
import jax.numpy as jnp
from jax import lax
from jax.experimental import pallas as pl
from jax.experimental.pallas import tpu as pltpu
from jax.experimental.pallas import tpu_sc as plsc

NC = 2
NS = 16
NW = NC * NS
C = 80
DW = 16

_SC_PARAMS = pltpu.CompilerParams(use_tc_tiling_on_sc=False)


def _mesh():
    return plsc.VectorSubcoreMesh(
        core_axis_name="c", subcore_axis_name="s",
        num_cores=NC, num_subcores=NS)


def _sc_deg(dstp, n_pad, chunks):
    rpt = n_pad // NS

    @functools.partial(
        pl.kernel,
        out_type=jax.ShapeDtypeStruct((NC, n_pad, DW), jnp.float32),
        mesh=_mesh(),
        compiler_params=_SC_PARAMS,
        scratch_types=[
            pltpu.VMEM_SHARED((n_pad, DW), jnp.float32),
            pltpu.VMEM((chunks, C), jnp.int32),
            pltpu.VMEM((C, DW), jnp.float32),
            pltpu.VMEM((rpt, DW), jnp.float32),
        ])
    def deg_kernel(dst_hbm, out_hbm, acc, dstv, ones_b, obuf):
        c = lax.axis_index("c")
        s = lax.axis_index("s")
        wid = s * NC + c
        pltpu.sync_copy(dst_hbm.at[wid], dstv)

        def fill_ones(i, carry):
            ones_b[i, :] = jnp.ones((DW,), jnp.float32)
            return carry

        lax.fori_loop(0, C, fill_ones, 0)

        def fill_zero(i, carry):
            obuf[i, :] = jnp.zeros((DW,), jnp.float32)
            return carry

        lax.fori_loop(0, rpt, fill_zero, 0)
        pltpu.sync_copy(obuf, acc.at[pl.ds(s * rpt, rpt)])
        plsc.subcore_barrier()

        def body(k, carry):
            pltpu.sync_copy(ones_b, acc.at[dstv.at[k]], add=True)
            return carry

        lax.fori_loop(0, chunks, body, 0)
        plsc.subcore_barrier()
        pltpu.sync_copy(acc.at[pl.ds(s * rpt, rpt)], obuf)
        pltpu.sync_copy(obuf, out_hbm.at[c, pl.ds(s * rpt, rpt)])

    return deg_kernel(dstp)


_R = 4
_H = 16


def _sc_scatter(y, srcp, dstp, n_pad, chunks, d_h):
    rpt = n_pad // NS

    @functools.partial(
        pl.kernel,
        out_type=jax.ShapeDtypeStruct((NC, n_pad, d_h), jnp.float32),
        mesh=_mesh(),
        compiler_params=_SC_PARAMS,
        scratch_types=[
            pltpu.VMEM_SHARED((n_pad, d_h), jnp.float32),
            pltpu.VMEM((2, _H, C), jnp.int32),
            pltpu.VMEM((2, _H, C), jnp.int32),
            [pltpu.VMEM((C, d_h), jnp.float32) for _ in range(_R)],
            [pltpu.SemaphoreType.DMA for _ in range(_R)],
            [pltpu.SemaphoreType.DMA for _ in range(_R)],
            pltpu.SemaphoreType.DMA,
        ])
    def scat_kernel(y_hbm, src_hbm, dst_hbm, out_hbm, acc,
                    srcv, dstv, rows, gsem, ssem, isem):
        c = lax.axis_index("c")
        s = lax.axis_index("s")
        wid = s * NC + c

        def load_half(half, start, sync):
            hh = min(_H, chunks - start)
            dsrc = pltpu.async_copy(src_hbm.at[wid, pl.ds(start, hh)],
                                    srcv.at[half, pl.ds(0, hh)], isem)
            ddst = pltpu.async_copy(dst_hbm.at[wid, pl.ds(start, hh)],
                                    dstv.at[half, pl.ds(0, hh)], isem)
            if sync:
                dsrc.wait()
                ddst.wait()
            return dsrc, ddst

        first_idx = load_half(0, 0, False)
        pend_idx = load_half(1, _H, False) if chunks > _H else None

        def fill_zero(i, carry):
            for j in range(d_h // 16):
                rows[0][i, pl.ds(j * 16, 16)] = jnp.zeros((16,), jnp.float32)
            return carry

        lax.fori_loop(0, C, fill_zero, 0)
        zcopies = []
        off = 0
        j = 0
        while off < rpt:
            sz = min(C, rpt - off)
            sem = gsem[j % _R] if j < _R else ssem[(j - _R) % _R]
            zcopies.append(pltpu.async_copy(
                rows[0].at[pl.ds(0, sz)],
                acc.at[pl.ds(s * rpt + off, sz)], sem))
            off += sz
            j += 1
        for d in zcopies:
            d.wait()
        first_idx[0].wait()
        first_idx[1].wait()
        plsc.subcore_barrier()

        def gather(k):
            return pltpu.async_copy(
                y_hbm.at[srcv.at[(k // _H) % 2, k % _H]],
                rows[k % _R], gsem[k % _R])

        ahead = _R - 2 if _R > 2 else 1
        d_g = [None] * _R
        d_s = [None] * _R
        for j in range(min(ahead, chunks)):
            d_g[j] = gather(j)
        for k in range(chunks):
            b = k % _R
            if k + ahead - _R >= 0:
                d_s[(k + ahead - _R) % _R].wait()
            if k % _H == 2 and k > _H:
                nxt_start = (k // _H + 1) * _H
                if nxt_start < chunks:
                    pend_idx = load_half((k // _H + 1) % 2, nxt_start, False)
            if k + ahead < chunks:
                if (k + ahead) % _H == 0 and pend_idx is not None:
                    pend_idx[0].wait()
                    pend_idx[1].wait()
                    pend_idx = None
                d_g[(k + ahead) % _R] = gather(k + ahead)
            d_g[b].wait()
            d_s[b] = pltpu.async_copy(
                rows[b], acc.at[dstv.at[(k // _H) % 2, k % _H]],
                ssem[b], add=True)
        for j in range(max(0, chunks + ahead - _R), chunks):
            d_s[j % _R].wait()
        plsc.subcore_barrier()
        pltpu.sync_copy(acc.at[pl.ds(s * rpt, rpt)],
                        out_hbm.at[c, pl.ds(s * rpt, rpt)])

    return scat_kernel(y, srcp, dstp)


_BR = 256


def _tc_b1(x_pad, W1, degp, n_pad, d_in, d_h):
    def kfn(x_ref, w_ref, d_ref, y_ref, dis_ref):
        dblk = d_ref[...]
        deg = dblk[0, :, 0] + dblk[1, :, 0] + 1.0
        dis = lax.rsqrt(deg)[:, None]
        xw = jnp.dot(x_ref[...], w_ref[...], preferred_element_type=jnp.float32)
        y_ref[...] = xw * dis
        dis_ref[...] = dis

    return pl.pallas_call(
        kfn,
        grid=(n_pad // _BR,),
        in_specs=[
            pl.BlockSpec((_BR, d_in), lambda i: (i, 0)),
            pl.BlockSpec((d_in, d_h), lambda i: (0, 0)),
            pl.BlockSpec((NC, _BR, DW), lambda i: (0, i, 0)),
        ],
        out_specs=[
            pl.BlockSpec((_BR, d_h), lambda i: (i, 0)),
            pl.BlockSpec((_BR, 1), lambda i: (i, 0)),
        ],
        out_shape=[
            jax.ShapeDtypeStruct((n_pad, d_h), jnp.float32),
            jax.ShapeDtypeStruct((n_pad, 1), jnp.float32),
        ],
    )(x_pad, W1, degp)


def _tc_mid(aggp, y1, dis, b1r, W2, n_pad, d_h):
    def kfn(a_ref, y_ref, d_ref, b_ref, w_ref, o_ref):
        a = a_ref[0] + a_ref[1] + y_ref[...]
        dis = d_ref[...]
        h = jnp.maximum(a * dis + b_ref[...], 0.0)
        o_ref[...] = jnp.dot(h, w_ref[...],
                             preferred_element_type=jnp.float32) * dis

    return pl.pallas_call(
        kfn,
        grid=(n_pad // _BR,),
        in_specs=[
            pl.BlockSpec((NC, _BR, d_h), lambda i: (0, i, 0)),
            pl.BlockSpec((_BR, d_h), lambda i: (i, 0)),
            pl.BlockSpec((_BR, 1), lambda i: (i, 0)),
            pl.BlockSpec((1, d_h), lambda i: (0, 0)),
            pl.BlockSpec((d_h, d_h), lambda i: (0, 0)),
        ],
        out_specs=pl.BlockSpec((_BR, d_h), lambda i: (i, 0)),
        out_shape=jax.ShapeDtypeStruct((n_pad, d_h), jnp.float32),
    )(aggp, y1, dis, b1r, W2)


def _tc_final(aggp, y2, dis, b2r, n_pad, d_h):
    def kfn(a_ref, y_ref, d_ref, b_ref, o_ref):
        a = a_ref[0] + a_ref[1] + y_ref[...]
        o_ref[...] = jnp.maximum(a * d_ref[...] + b_ref[...], 0.0)

    return pl.pallas_call(
        kfn,
        grid=(n_pad // _BR,),
        in_specs=[
            pl.BlockSpec((NC, _BR, d_h), lambda i: (0, i, 0)),
            pl.BlockSpec((_BR, d_h), lambda i: (i, 0)),
            pl.BlockSpec((_BR, 1), lambda i: (i, 0)),
            pl.BlockSpec((1, d_h), lambda i: (0, 0)),
        ],
        out_specs=pl.BlockSpec((_BR, d_h), lambda i: (i, 0)),
        out_shape=jax.ShapeDtypeStruct((n_pad, d_h), jnp.float32),
    )(aggp, y2, dis, b2r)


def kernel(x, edge_index, W1, b1, W2, b2):
    n, d_in = x.shape
    e = edge_index.shape[1]
    d_h = W1.shape[1]

    chunks = -(-e // (NW * C))
    e_pad = NW * chunks * C
    rpt = 128 * (-(-(n + 1) // (NS * 128)))
    n_pad = NS * rpt

    src = jnp.pad(edge_index[0], (0, e_pad - e), constant_values=n)
    dst = jnp.pad(edge_index[1], (0, e_pad - e), constant_values=n)
    srcp = src.reshape(NW, chunks, C)
    dstp = dst.reshape(NW, chunks, C)
    x_pad = jnp.pad(x, ((0, n_pad - n), (0, 0)))
    b1r = b1.reshape(1, d_h)
    b2r = b2.reshape(1, d_h)

    degp = _sc_deg(dstp, n_pad, chunks)
    y1, dis = _tc_b1(x_pad, W1, degp, n_pad, d_in, d_h)
    agg1 = _sc_scatter(y1, srcp, dstp, n_pad, chunks, d_h)
    y2 = _tc_mid(agg1, y1, dis, b1r, W2, n_pad, d_h)
    agg2 = _sc_scatter(y2, srcp, dstp, n_pad, chunks, d_h)
    h2 = _tc_final(agg2, y2, dis, b2r, n_pad, d_h)
    return h2[:n]


# speedup vs baseline: 1.0381x; 1.0151x over previous
import functools

import jax
import jax.numpy as jnp
from jax import lax
from jax.experimental import pallas as pl
from jax.experimental.pallas import tpu as pltpu
from jax.experimental.pallas import tpu_sc as plsc

NC = 2
NS = 16
NW = NC * NS
C = 80
DW = 16

_SC_PARAMS = pltpu.CompilerParams(use_tc_tiling_on_sc=False)


def _mesh():
    return plsc.VectorSubcoreMesh(
        core_axis_name="c", subcore_axis_name="s",
        num_cores=NC, num_subcores=NS)


def _sc_deg(dstp, n_pad, chunks):
    rpt = n_pad // NS

    @functools.partial(
        pl.kernel,
        out_type=jax.ShapeDtypeStruct((NC, n_pad, DW), jnp.float32),
        mesh=_mesh(),
        compiler_params=_SC_PARAMS,
        scratch_types=[
            pltpu.VMEM_SHARED((n_pad, DW), jnp.float32),
            pltpu.VMEM((chunks, C), jnp.int32),
            pltpu.VMEM((C, DW), jnp.float32),
            pltpu.VMEM((rpt, DW), jnp.float32),
            [pltpu.SemaphoreType.DMA for _ in range(4)],
        ])
    def deg_kernel(dst_hbm, out_hbm, acc, dstv, ones_b, obuf, dsem):
        c = lax.axis_index("c")
        s = lax.axis_index("s")
        wid = s * NC + c
        pltpu.sync_copy(dst_hbm.at[wid], dstv)

        def fill_ones(i, carry):
            ones_b[i, :] = jnp.ones((DW,), jnp.float32)
            return carry

        lax.fori_loop(0, C, fill_ones, 0)

        def fill_zero(i, carry):
            obuf[i, :] = jnp.zeros((DW,), jnp.float32)
            return carry

        lax.fori_loop(0, rpt, fill_zero, 0)
        pltpu.sync_copy(obuf, acc.at[pl.ds(s * rpt, rpt)])
        plsc.subcore_barrier()

        d = [None] * 4
        for k in range(chunks):
            b = k % 4
            if d[b] is not None:
                d[b].wait()
            d[b] = pltpu.async_copy(ones_b, acc.at[dstv.at[k]], dsem[b],
                                    add=True)
        for b in range(4):
            if d[b] is not None:
                d[b].wait()
        plsc.subcore_barrier()
        pltpu.sync_copy(acc.at[pl.ds(s * rpt, rpt)], obuf)
        pltpu.sync_copy(obuf, out_hbm.at[c, pl.ds(s * rpt, rpt)])

    return deg_kernel(dstp)


_R = 4
_H = 16


def _sc_scatter(y, srcp, dstp, n_pad, chunks, d_h):
    rpt = n_pad // NS

    @functools.partial(
        pl.kernel,
        out_type=jax.ShapeDtypeStruct((NC, n_pad, d_h), jnp.float32),
        mesh=_mesh(),
        compiler_params=_SC_PARAMS,
        scratch_types=[
            pltpu.VMEM_SHARED((n_pad, d_h), jnp.float32),
            pltpu.VMEM((2, _H, C), jnp.int32),
            pltpu.VMEM((2, _H, C), jnp.int32),
            [pltpu.VMEM((C, d_h), jnp.float32) for _ in range(_R)],
            [pltpu.SemaphoreType.DMA for _ in range(_R)],
            [pltpu.SemaphoreType.DMA for _ in range(_R)],
            pltpu.SemaphoreType.DMA,
        ])
    def scat_kernel(y_hbm, src_hbm, dst_hbm, out_hbm, acc,
                    srcv, dstv, rows, gsem, ssem, isem):
        c = lax.axis_index("c")
        s = lax.axis_index("s")
        wid = s * NC + c

        def load_half(half, start, sync):
            hh = min(_H, chunks - start)
            dsrc = pltpu.async_copy(src_hbm.at[wid, pl.ds(start, hh)],
                                    srcv.at[half, pl.ds(0, hh)], isem)
            ddst = pltpu.async_copy(dst_hbm.at[wid, pl.ds(start, hh)],
                                    dstv.at[half, pl.ds(0, hh)], isem)
            if sync:
                dsrc.wait()
                ddst.wait()
            return dsrc, ddst

        first_idx = load_half(0, 0, False)
        pend_idx = load_half(1, _H, False) if chunks > _H else None

        def fill_zero(i, carry):
            for j in range(d_h // 16):
                rows[0][i, pl.ds(j * 16, 16)] = jnp.zeros((16,), jnp.float32)
            return carry

        lax.fori_loop(0, C, fill_zero, 0)
        zcopies = []
        off = 0
        j = 0
        while off < rpt:
            sz = min(C, rpt - off)
            sem = gsem[j % _R] if j < _R else ssem[(j - _R) % _R]
            zcopies.append(pltpu.async_copy(
                rows[0].at[pl.ds(0, sz)],
                acc.at[pl.ds(s * rpt + off, sz)], sem))
            off += sz
            j += 1
        for d in zcopies:
            d.wait()
        first_idx[0].wait()
        first_idx[1].wait()
        plsc.subcore_barrier()

        def gather(k):
            return pltpu.async_copy(
                y_hbm.at[srcv.at[(k // _H) % 2, k % _H]],
                rows[k % _R], gsem[k % _R])

        ahead = _R - 2 if _R > 2 else 1
        d_g = [None] * _R
        d_s = [None] * _R
        for j in range(min(ahead, chunks)):
            d_g[j] = gather(j)
        for k in range(chunks):
            b = k % _R
            if k + ahead - _R >= 0:
                d_s[(k + ahead - _R) % _R].wait()
            if k % _H == 2 and k > _H:
                nxt_start = (k // _H + 1) * _H
                if nxt_start < chunks:
                    pend_idx = load_half((k // _H + 1) % 2, nxt_start, False)
            if k + ahead < chunks:
                if (k + ahead) % _H == 0 and pend_idx is not None:
                    pend_idx[0].wait()
                    pend_idx[1].wait()
                    pend_idx = None
                d_g[(k + ahead) % _R] = gather(k + ahead)
            d_g[b].wait()
            d_s[b] = pltpu.async_copy(
                rows[b], acc.at[dstv.at[(k // _H) % 2, k % _H]],
                ssem[b], add=True)
        for j in range(max(0, chunks + ahead - _R), chunks):
            d_s[j % _R].wait()
        plsc.subcore_barrier()
        pltpu.sync_copy(acc.at[pl.ds(s * rpt, rpt)],
                        out_hbm.at[c, pl.ds(s * rpt, rpt)])

    return scat_kernel(y, srcp, dstp)


_BR = 256


def _tc_b1(x_pad, W1, degp, n_pad, d_in, d_h):
    def kfn(x_ref, w_ref, d_ref, y_ref, dis_ref):
        dblk = d_ref[...]
        deg = dblk[0, :, 0] + dblk[1, :, 0] + 1.0
        dis = lax.rsqrt(deg)[:, None]
        xw = jnp.dot(x_ref[...], w_ref[...], preferred_element_type=jnp.float32)
        y_ref[...] = xw * dis
        dis_ref[...] = dis

    return pl.pallas_call(
        kfn,
        grid=(n_pad // _BR,),
        in_specs=[
            pl.BlockSpec((_BR, d_in), lambda i: (i, 0)),
            pl.BlockSpec((d_in, d_h), lambda i: (0, 0)),
            pl.BlockSpec((NC, _BR, DW), lambda i: (0, i, 0)),
        ],
        out_specs=[
            pl.BlockSpec((_BR, d_h), lambda i: (i, 0)),
            pl.BlockSpec((_BR, 1), lambda i: (i, 0)),
        ],
        out_shape=[
            jax.ShapeDtypeStruct((n_pad, d_h), jnp.float32),
            jax.ShapeDtypeStruct((n_pad, 1), jnp.float32),
        ],
    )(x_pad, W1, degp)


def _tc_mid(aggp, y1, dis, b1r, W2, n_pad, d_h):
    def kfn(a_ref, y_ref, d_ref, b_ref, w_ref, o_ref):
        a = a_ref[0] + a_ref[1] + y_ref[...]
        dis = d_ref[...]
        h = jnp.maximum(a * dis + b_ref[...], 0.0)
        o_ref[...] = jnp.dot(h, w_ref[...],
                             preferred_element_type=jnp.float32) * dis

    return pl.pallas_call(
        kfn,
        grid=(n_pad // _BR,),
        in_specs=[
            pl.BlockSpec((NC, _BR, d_h), lambda i: (0, i, 0)),
            pl.BlockSpec((_BR, d_h), lambda i: (i, 0)),
            pl.BlockSpec((_BR, 1), lambda i: (i, 0)),
            pl.BlockSpec((1, d_h), lambda i: (0, 0)),
            pl.BlockSpec((d_h, d_h), lambda i: (0, 0)),
        ],
        out_specs=pl.BlockSpec((_BR, d_h), lambda i: (i, 0)),
        out_shape=jax.ShapeDtypeStruct((n_pad, d_h), jnp.float32),
    )(aggp, y1, dis, b1r, W2)


def _tc_final(aggp, y2, dis, b2r, n_pad, d_h):
    def kfn(a_ref, y_ref, d_ref, b_ref, o_ref):
        a = a_ref[0] + a_ref[1] + y_ref[...]
        o_ref[...] = jnp.maximum(a * d_ref[...] + b_ref[...], 0.0)

    return pl.pallas_call(
        kfn,
        grid=(n_pad // _BR,),
        in_specs=[
            pl.BlockSpec((NC, _BR, d_h), lambda i: (0, i, 0)),
            pl.BlockSpec((_BR, d_h), lambda i: (i, 0)),
            pl.BlockSpec((_BR, 1), lambda i: (i, 0)),
            pl.BlockSpec((1, d_h), lambda i: (0, 0)),
        ],
        out_specs=pl.BlockSpec((_BR, d_h), lambda i: (i, 0)),
        out_shape=jax.ShapeDtypeStruct((n_pad, d_h), jnp.float32),
    )(aggp, y2, dis, b2r)


def kernel(x, edge_index, W1, b1, W2, b2):
    n, d_in = x.shape
    e = edge_index.shape[1]
    d_h = W1.shape[1]

    chunks = -(-e // (NW * C))
    e_pad = NW * chunks * C
    rpt = 128 * (-(-(n + 1) // (NS * 128)))
    n_pad = NS * rpt

    src = jnp.pad(edge_index[0], (0, e_pad - e), constant_values=n)
    dst = jnp.pad(edge_index[1], (0, e_pad - e), constant_values=n)
    srcp = src.reshape(NW, chunks, C)
    dstp = dst.reshape(NW, chunks, C)
    x_pad = jnp.pad(x, ((0, n_pad - n), (0, 0)))
    b1r = b1.reshape(1, d_h)
    b2r = b2.reshape(1, d_h)

    degp = _sc_deg(dstp, n_pad, chunks)
    y1, dis = _tc_b1(x_pad, W1, degp, n_pad, d_in, d_h)
    agg1 = _sc_scatter(y1, srcp, dstp, n_pad, chunks, d_h)
    y2 = _tc_mid(agg1, y1, dis, b1r, W2, n_pad, d_h)
    agg2 = _sc_scatter(y2, srcp, dstp, n_pad, chunks, d_h)
    h2 = _tc_final(agg2, y2, dis, b2r, n_pad, d_h)
    return h2[:n]
